# baseline (speedup 1.0000x reference)
import jax
import jax.numpy as jnp
from jax import lax
from jax.experimental import pallas as pl
from jax.experimental.pallas import tpu as pltpu
from jax.experimental.pallas import tpu_sc as plsc

B = 4096
S = 50
H = S // 2
D = 64
L = 16
NC = 2
NS = 16
NW = NC * NS
B_PER_W = B // NW
HB = B_PER_W // 2
SP = 64


def kernel(indices, table):
    idx = indices.astype(jnp.int32)
    idx3 = jnp.pad(idx, ((0, 0), (0, SP - S))).reshape(NW, B_PER_W, SP)
    mesh = plsc.VectorSubcoreMesh(core_axis_name="c", subcore_axis_name="s")

    @pl.kernel(
        out_type=jax.ShapeDtypeStruct((B, D), jnp.float32),
        mesh=mesh,
        scratch_types=[
            pltpu.VMEM((B_PER_W, SP), jnp.int32),
            pltpu.VMEM((H, 8, D), jnp.float32),
            pltpu.VMEM((H, 8, D), jnp.float32),
            pltpu.VMEM((H, 8, D), jnp.float32),
            pltpu.VMEM((H, 8, D), jnp.float32),
            pltpu.VMEM((HB, D), jnp.float32),
            pltpu.SemaphoreType.DMA,
            pltpu.SemaphoreType.DMA,
            pltpu.SemaphoreType.DMA,
            pltpu.SemaphoreType.DMA,
        ],
        compiler_params=pltpu.CompilerParams(use_tc_tiling_on_sc=True),
    )
    def sc_kernel(table_hbm, idx_hbm, out_hbm, idx_v,
                  buf_a0, buf_a1, buf_b0, buf_b1, out_v,
                  sem_a0, sem_a1, sem_b0, sem_b1):
        wid = lax.axis_index("s") * NC + lax.axis_index("c")
        pltpu.sync_copy(idx_hbm.at[wid], idx_v)

        def start(b, phase, buf, sem):
            qvecs = [(idx_v[b, pl.ds(k * L, L)] >> 3) << 3 for k in range(4)]
            for j in range(H):
                t = phase * H + j
                q8 = pl.multiple_of(qvecs[t // L][t % L], 8)
                pltpu.async_copy(table_hbm.at[pl.ds(q8, 8)], buf.at[j], sem)

        def wait(buf, sem):
            for j in range(H):
                pltpu.make_async_copy(
                    table_hbm.at[pl.ds(0, 8)], buf.at[j], sem).wait()

        def accumulate(buf, b, phase):
            rvecs = [idx_v[b, pl.ds(k * L, L)] & 7 for k in range(4)]
            accs = [None] * (D // L)
            for j in range(H):
                t = phase * H + j
                rr = rvecs[t // L][t % L]
                for d in range(D // L):
                    sl = pl.ds(d * L, L)
                    v = buf[j, rr, sl]
                    accs[d] = v if accs[d] is None else accs[d] + v
            bb = b & (HB - 1)
            for d in range(D // L):
                sl = pl.ds(d * L, L)
                if phase == 0:
                    out_v[bb, sl] = accs[d]
                else:
                    out_v[bb, sl] = (out_v[bb, sl] + accs[d]) * (1.0 / S)

        start(0, 0, buf_a0, sem_a0)
        start(0, 1, buf_a1, sem_a1)
        start(1, 0, buf_b0, sem_b0)
        start(1, 1, buf_b1, sem_b1)

        @pl.loop(0, B_PER_W // 2)
        def _(g):
            b0 = 2 * g
            b1 = b0 + 1

            def item(b, phase, buf, sem):
                wait(buf, sem)
                accumulate(buf, b, phase)
                @pl.when(b + 2 < B_PER_W)
                def _():
                    start(b + 2, phase, buf, sem)

            item(b0, 0, buf_a0, sem_a0)
            item(b0, 1, buf_a1, sem_a1)
            item(b1, 0, buf_b0, sem_b0)
            item(b1, 1, buf_b1, sem_b1)
            @pl.when(b1 == HB - 1)
            def _():
                pltpu.sync_copy(out_v, out_hbm.at[pl.ds(wid * B_PER_W, HB)])

        pltpu.sync_copy(out_v, out_hbm.at[pl.ds(wid * B_PER_W + HB, HB)])

    return sc_kernel(table, idx3)
